# baseline (speedup 1.0000x reference)
import functools

import jax
import jax.numpy as jnp
from jax import lax
from jax.experimental import pallas as pl
from jax.experimental.pallas import tpu as pltpu
from jax.experimental.pallas import tpu_sc as plsc


def _splat(vreg, lane):
    return jnp.broadcast_to(lax.slice(vreg, (lane,), (lane + 1,)), (16,))


def _sc_body(V, D, H, x_hbm, emb_hbm, w1_hbm, b1_hbm, w2_hbm, b2_hbm,
             out_hbm, idx_v, rows_v, ec_v, w1c_v, b1c_v, hpub_v, h_v,
             w2c_v, b2c_v, out_v, ec_sh, h_sh, sem, sem2):
    cid = lax.axis_index("c")
    sid = lax.axis_index("s")
    on_core0 = cid == 0
    it = lax.iota(jnp.int32, 16)
    n_o = V // 16 + 1

    @pl.when(on_core0 & (sid < n_o))
    def _prefetch():
        pltpu.make_async_copy(
            w2_hbm.at[pl.ds(sid * 16 * H, 16 * H)], w2c_v, sem2).start()

    @pl.when(on_core0 & (sid == 0))
    def _gather():
        idx_v[...] = jnp.zeros((16,), jnp.int32)
        pltpu.sync_copy(x_hbm, idx_v.at[pl.ds(0, 2)])
        pltpu.async_copy(emb_hbm.at[idx_v], rows_v, sem).wait()
        pltpu.sync_copy(rows_v.at[pl.ds(0, 2)], ec_sh)

    plsc.subcore_barrier()

    def _h_chunk(c):
        pltpu.sync_copy(w1_hbm.at[pl.ds(c * 16 * 2 * D, 16 * 2 * D)], w1c_v)
        pltpu.sync_copy(b1_hbm.at[pl.ds(c * 16, 16)], b1c_v)
        ec = [(ec_v[r, pl.ds(0, 16)], ec_v[r, pl.ds(D - 16, 16)])
              for r in range(2)]
        acc = b1c_v[...]
        for k in range(2 * D):
            r, d = k // D, k % D
            if d < 16:
                s = _splat(ec[r][0], d)
            else:
                s = _splat(ec[r][1], d - (D - 16))
            acc = acc + s * w1c_v[pl.ds(k * 16, 16)]
        acc = jnp.maximum(acc, 0.0)
        acc = jnp.where(it < H - c * 16, acc, 0.0)
        hpub_v[...] = acc
        pltpu.sync_copy(hpub_v, h_sh.at[pl.ds(c * 16, 16)])

    @pl.when(on_core0)
    def _stage1():
        pltpu.sync_copy(ec_sh, ec_v)
        _h_chunk(sid)

        @pl.when(sid < (H + 15) // 16 - 16)
        def _round2():
            _h_chunk(sid + 16)

    plsc.subcore_barrier()

    @pl.when(on_core0 & (sid < n_o))
    def _stage2():
        pltpu.sync_copy(h_sh, h_v)
        pltpu.sync_copy(b2_hbm.at[pl.ds(sid * 16, 16)], b2c_v)
        pltpu.make_async_copy(
            w2_hbm.at[pl.ds(sid * 16 * H, 16 * H)], w2c_v, sem2).wait()
        hq = [h_v[pl.ds(q * 16, 16)] for q in range((H + 15) // 16)]
        acc = b2c_v[...]
        for j in range(H):
            s = _splat(hq[j // 16], j % 16)
            acc = acc + s * w2c_v[pl.ds(j * 16, 16)]
        out_v[...] = acc
        pltpu.sync_copy(out_v, out_hbm.at[pl.ds(sid * 16, 16)])


def kernel(x, embed, W1, b1, W2, b2):
    V, D = embed.shape
    H = W1.shape[0]
    n_h = (H + 15) // 16
    n_o = (V + 15) // 16
    Hp, Vp = n_h * 16, n_o * 16
    embp = jnp.pad(embed, ((0, 0), (0, 128 - D)))
    w1f = jnp.pad(W1, ((0, Hp - H), (0, 0))).reshape(n_h, 16, 2 * D)
    w1f = w1f.transpose(0, 2, 1).reshape(-1)
    w2f = jnp.pad(W2, ((0, Vp - V), (0, 0))).reshape(n_o, 16, H)
    w2f = w2f.transpose(0, 2, 1).reshape(-1)
    b1f = jnp.pad(b1, (0, Hp - H))
    b2f = jnp.pad(b2, (0, Vp - V))

    mesh = plsc.VectorSubcoreMesh(core_axis_name="c", subcore_axis_name="s")
    body = functools.partial(_sc_body, V, D, H)
    k = pl.kernel(
        body,
        mesh=mesh,
        out_type=jax.ShapeDtypeStruct((Vp,), jnp.float32),
        scratch_types=[
            pltpu.VMEM((16,), jnp.int32),
            pltpu.VMEM((16, 128), jnp.float32),
            pltpu.VMEM((2, 128), jnp.float32),
            pltpu.VMEM((16 * 2 * D,), jnp.float32),
            pltpu.VMEM((16,), jnp.float32),
            pltpu.VMEM((16,), jnp.float32),
            pltpu.VMEM((Hp,), jnp.float32),
            pltpu.VMEM((16 * H,), jnp.float32),
            pltpu.VMEM((16,), jnp.float32),
            pltpu.VMEM((16,), jnp.float32),
            pltpu.VMEM_SHARED((2, 128), jnp.float32),
            pltpu.VMEM_SHARED((Hp,), jnp.float32),
            pltpu.SemaphoreType.DMA,
            pltpu.SemaphoreType.DMA,
        ],
    )
    out = k(x.astype(jnp.int32), embp, w1f, b1f, w2f, b2f)
    return out[:V].reshape(1, V)
